# Initial kernel scaffold; baseline (speedup 1.0000x reference)
import jax
import jax.numpy as jnp
from jax.experimental import pallas as pl


def kernel(x, pos_table):
    raise NotImplementedError("write your pallas kernel here")


# speedup vs baseline: 1.6680x; 1.6680x over previous
import jax
import jax.numpy as jnp
from jax.experimental import pallas as pl

_SEQ_TILE = 1024


def _add_kernel(x_ref, pos_ref, out_ref):
    out_ref[0] = x_ref[0] + pos_ref[...]


def kernel(x, pos_table):
    batch, seq, n_embd = x.shape
    grid = (seq // _SEQ_TILE, batch)
    return pl.pallas_call(
        _add_kernel,
        grid=grid,
        in_specs=[
            pl.BlockSpec((1, _SEQ_TILE, n_embd), lambda s, b: (b, s, 0)),
            pl.BlockSpec((_SEQ_TILE, n_embd), lambda s, b: (s, 0)),
        ],
        out_specs=pl.BlockSpec((1, _SEQ_TILE, n_embd), lambda s, b: (b, s, 0)),
        out_shape=jax.ShapeDtypeStruct(x.shape, x.dtype),
    )(x, pos_table)


# speedup vs baseline: 1.7404x; 1.0434x over previous
import jax
import jax.numpy as jnp
from jax.experimental import pallas as pl

_SEQ_TILE = 2048


def _add_kernel(x_ref, pos_ref, out_ref):
    out_ref[0] = x_ref[0] + pos_ref[...]


def kernel(x, pos_table):
    batch, seq, n_embd = x.shape
    grid = (seq // _SEQ_TILE, batch)
    return pl.pallas_call(
        _add_kernel,
        grid=grid,
        in_specs=[
            pl.BlockSpec((1, _SEQ_TILE, n_embd), lambda s, b: (b, s, 0)),
            pl.BlockSpec((_SEQ_TILE, n_embd), lambda s, b: (s, 0)),
        ],
        out_specs=pl.BlockSpec((1, _SEQ_TILE, n_embd), lambda s, b: (b, s, 0)),
        out_shape=jax.ShapeDtypeStruct(x.shape, x.dtype),
    )(x, pos_table)
